# Initial kernel scaffold; baseline (speedup 1.0000x reference)
import jax
import jax.numpy as jnp
from jax.experimental import pallas as pl


def kernel(x, edge_index, w0_1, W1, b1, w0_2, W2, b2):
    raise NotImplementedError("write your pallas kernel here")


# speedup vs baseline: 30.5283x; 30.5283x over previous
import functools

import jax
import jax.numpy as jnp
from jax import lax
from jax.experimental import pallas as pl
from jax.experimental.pallas import tpu as pltpu
from jax.experimental.pallas import tpu_sc as plsc

N = 10000
E = 320000
D_IN = 128
H = 16
C = 40

NP = 10240
F2 = 48

NC = 2
NS = 16
NW = NC * NS
EPW = E // NW
K = 125
CH = EPW // K
RP = NP // NS

_mesh = plsc.VectorSubcoreMesh(core_axis_name="c", subcore_axis_name="s")
_sc_params = pltpu.CompilerParams(use_tc_tiling_on_sc=False)


def _sc_deg(col2d, zeros16):
  @functools.partial(
      pl.kernel,
      out_type=jax.ShapeDtypeStruct((NC, NP, H), jnp.float32),
      mesh=_mesh,
      compiler_params=_sc_params,
      scratch_types=[
          pltpu.VMEM((CH, K), jnp.int32),
          pltpu.VMEM((K, H), jnp.float32),
          pltpu.VMEM((RP, H), jnp.float32),
          pltpu.VMEM_SHARED((NP, H), jnp.float32),
      ],
  )
  def k(col_hbm, z_hbm, out_hbm, idx_v, ones_v, zero_v, acc_sp):
    c = lax.axis_index("c")
    s = lax.axis_index("s")
    w = s * NC + c

    def fill_ones(i, _):
      ones_v[i, :] = jnp.ones((H,), jnp.float32)
      return 0

    lax.fori_loop(0, K, fill_ones, 0)

    pltpu.sync_copy(z_hbm.at[pl.ds(s * RP, RP)], zero_v)
    pltpu.sync_copy(zero_v, acc_sp.at[pl.ds(s * RP, RP)])
    plsc.subcore_barrier()

    pltpu.sync_copy(col_hbm.at[pl.ds(w * CH, CH)], idx_v)

    def body(j, _):
      pltpu.sync_copy(ones_v, acc_sp.at[idx_v.at[j]], add=True)
      return 0

    lax.fori_loop(0, CH, body, 0)
    plsc.subcore_barrier()
    pltpu.sync_copy(acc_sp.at[pl.ds(s * RP, RP)],
                    out_hbm.at[c, pl.ds(s * RP, RP)])

  return k(col2d, zeros16)


def _sc_agg(row2d, col2d, table, zeros, f):
  @functools.partial(
      pl.kernel,
      out_type=jax.ShapeDtypeStruct((NC, NP, f), jnp.float32),
      mesh=_mesh,
      compiler_params=_sc_params,
      scratch_types=[
          pltpu.VMEM((CH, K), jnp.int32),
          pltpu.VMEM((CH, K), jnp.int32),
          pltpu.VMEM((K, f), jnp.float32),
          pltpu.VMEM((RP, f), jnp.float32),
          pltpu.VMEM_SHARED((NP, f), jnp.float32),
      ],
  )
  def k(row_hbm, col_hbm, tab_hbm, z_hbm, out_hbm,
        idxr_v, idxc_v, buf, zero_v, acc_sp):
    c = lax.axis_index("c")
    s = lax.axis_index("s")
    w = s * NC + c

    pltpu.sync_copy(z_hbm.at[pl.ds(s * RP, RP)], zero_v)
    pltpu.sync_copy(zero_v, acc_sp.at[pl.ds(s * RP, RP)])
    plsc.subcore_barrier()

    pltpu.sync_copy(row_hbm.at[pl.ds(w * CH, CH)], idxr_v)
    pltpu.sync_copy(col_hbm.at[pl.ds(w * CH, CH)], idxc_v)

    def body(j, _):
      pltpu.sync_copy(tab_hbm.at[idxr_v.at[j]], buf)
      pltpu.sync_copy(buf, acc_sp.at[idxc_v.at[j]], add=True)
      return 0

    lax.fori_loop(0, CH, body, 0)
    plsc.subcore_barrier()
    pltpu.sync_copy(acc_sp.at[pl.ds(s * RP, RP)],
                    out_hbm.at[c, pl.ds(s * RP, RP)])

  return k(row2d, col2d, table, zeros)


def _tc1(x_ref, w_ref, o_ref):
  o_ref[...] = jnp.dot(x_ref[...], w_ref[...],
                       preferred_element_type=jnp.float32)


def _tc2(degp_ref, h1_ref, h1p_ref, dis_ref):
  d = degp_ref[0] + degp_ref[1]
  deg = d[:, 0:1]
  dis = jnp.where(deg > 0, lax.rsqrt(jnp.maximum(deg, 1e-12)), 0.0)
  dis_ref[...] = dis
  h1p_ref[...] = h1_ref[...] * dis


def _tc3(dis_ref, aggp_ref, h1_ref, w0_ref, b1_ref, w2_ref,
         h2_ref, h2p_ref):
  dis = dis_ref[...]
  agg = aggp_ref[0] + aggp_ref[1]
  z = dis * agg + jax.nn.sigmoid(w0_ref[...]) * h1_ref[...] + b1_ref[...]
  z = jnp.maximum(z, 0.0)
  h2 = jnp.dot(z, w2_ref[...], preferred_element_type=jnp.float32)
  h2_ref[...] = h2
  h2p_ref[...] = h2 * dis


def _tc4(dis_ref, aggp_ref, h2_ref, w0_ref, b2_ref, o_ref):
  agg = aggp_ref[0] + aggp_ref[1]
  o_ref[...] = (dis_ref[...] * agg
                + jax.nn.sigmoid(w0_ref[...]) * h2_ref[...] + b2_ref[...])


def kernel(x, edge_index, w0_1, W1, b1, w0_2, W2, b2):
  ei = edge_index.astype(jnp.int32)
  row2d = ei[0].reshape(NW * CH, K)
  col2d = ei[1].reshape(NW * CH, K)

  x_pad = jnp.pad(x, ((0, NP - N), (0, 0)))
  w0_1p = jnp.pad(w0_1, (0, NP - N)).reshape(NP, 1)
  w0_2p = jnp.pad(w0_2, (0, NP - N)).reshape(NP, 1)
  W2p = jnp.pad(W2, ((0, 0), (0, F2 - C)))
  b1r = b1.reshape(1, H)
  b2r = jnp.pad(b2, (0, F2 - C)).reshape(1, F2)
  zeros48 = jnp.zeros((NP, F2), jnp.float32)
  zeros16 = zeros48[:, :H]

  f32 = jnp.float32
  degp = _sc_deg(col2d, zeros16)
  h1 = pl.pallas_call(
      _tc1, out_shape=jax.ShapeDtypeStruct((NP, H), f32))(x_pad, W1)
  h1p, dis = pl.pallas_call(
      _tc2, out_shape=(jax.ShapeDtypeStruct((NP, H), f32),
                       jax.ShapeDtypeStruct((NP, 1), f32)))(degp, h1)
  agg1p = _sc_agg(row2d, col2d, h1p, zeros16, H)
  h2, h2p = pl.pallas_call(
      _tc3, out_shape=(jax.ShapeDtypeStruct((NP, F2), f32),
                       jax.ShapeDtypeStruct((NP, F2), f32)))(
          dis, agg1p, h1, w0_1p, b1r, W2p)
  agg2p = _sc_agg(row2d, col2d, h2p, zeros48, F2)
  out = pl.pallas_call(
      _tc4, out_shape=jax.ShapeDtypeStruct((NP, F2), f32))(
          dis, agg2p, h2, w0_2p, b2r)
  return out[:N, :C]


# speedup vs baseline: 43.2960x; 1.4182x over previous
import functools

import jax
import jax.numpy as jnp
from jax import lax
from jax.experimental import pallas as pl
from jax.experimental.pallas import tpu as pltpu
from jax.experimental.pallas import tpu_sc as plsc

N = 10000
E = 320000
D_IN = 128
H = 16
C = 40

NP = 10240
F2 = 48

NC = 2
NS = 16
NW = NC * NS
EPW = E // NW
K = 125
CH = EPW // K
CHQ = CH // 4
RP = NP // NS
DEG_Q = 16

_mesh = plsc.VectorSubcoreMesh(core_axis_name="c", subcore_axis_name="s")
_sc_params = pltpu.CompilerParams(use_tc_tiling_on_sc=False)


def _sc_deg(col2d, zeros16):
  @functools.partial(
      pl.kernel,
      out_type=jax.ShapeDtypeStruct((NC, NP, H), jnp.float32),
      mesh=_mesh,
      compiler_params=_sc_params,
      scratch_types=[
          pltpu.VMEM((CH, K), jnp.int32),
          pltpu.VMEM((K, H), jnp.float32),
          pltpu.VMEM((RP, H), jnp.float32),
          pltpu.VMEM_SHARED((NP, H), jnp.float32),
          pltpu.SemaphoreType.DMA,
          pltpu.SemaphoreType.DMA,
      ],
  )
  def k(col_hbm, z_hbm, out_hbm, idx_v, ones_v, zero_v, acc_sp, semi, sem):
    c = lax.axis_index("c")
    s = lax.axis_index("s")
    w = s * NC + c

    ld = pltpu.async_copy(col_hbm.at[pl.ds(w * CH, CH)], idx_v, semi)

    def fill_ones(i, _):
      ones_v[i, :] = jnp.ones((H,), jnp.float32)
      return 0

    lax.fori_loop(0, K, fill_ones, 0)
    pltpu.sync_copy(z_hbm.at[pl.ds(s * RP, RP)], zero_v)
    pltpu.sync_copy(zero_v, acc_sp.at[pl.ds(s * RP, RP)])
    ld.wait()
    plsc.subcore_barrier()

    def wait_one():
      pltpu.make_async_copy(ones_v, acc_sp.at[pl.ds(0, K)], sem).wait()

    def body(j, _):
      pltpu.async_copy(ones_v, acc_sp.at[idx_v.at[j]], sem, add=True)

      @pl.when(j >= DEG_Q)
      def _():
        wait_one()

      return 0

    lax.fori_loop(0, CH, body, 0)

    def drain(j, _):
      wait_one()
      return 0

    lax.fori_loop(0, DEG_Q, drain, 0)
    plsc.subcore_barrier()
    pltpu.sync_copy(acc_sp.at[pl.ds(s * RP, RP)],
                    out_hbm.at[c, pl.ds(s * RP, RP)])

  return k(col2d, zeros16)


def _sc_agg(row2d, col2d, table, zeros, f):
  @functools.partial(
      pl.kernel,
      out_type=jax.ShapeDtypeStruct((NC, NP, f), jnp.float32),
      mesh=_mesh,
      compiler_params=_sc_params,
      scratch_types=[
          pltpu.VMEM((CH, K), jnp.int32),
          pltpu.VMEM((CH, K), jnp.int32),
          pltpu.VMEM((4, K, f), jnp.float32),
          pltpu.VMEM((RP, f), jnp.float32),
          pltpu.VMEM_SHARED((NP, f), jnp.float32),
      ] + [pltpu.SemaphoreType.DMA] * 8,
  )
  def k(row_hbm, col_hbm, tab_hbm, z_hbm, out_hbm,
        idxr_v, idxc_v, buf, zero_v, acc_sp, *sems8):
    semg = sems8[0:4]
    sems = sems8[4:8]
    c = lax.axis_index("c")
    s = lax.axis_index("s")
    w = s * NC + c

    ldr = pltpu.async_copy(row_hbm.at[pl.ds(w * CH, CH)], idxr_v, semg[0])
    ldc = pltpu.async_copy(col_hbm.at[pl.ds(w * CH, CH)], idxc_v, semg[1])
    pltpu.sync_copy(z_hbm.at[pl.ds(s * RP, RP)], zero_v)
    pltpu.sync_copy(zero_v, acc_sp.at[pl.ds(s * RP, RP)])
    ldr.wait()
    ldc.wait()
    plsc.subcore_barrier()

    def g(j, q):
      pltpu.async_copy(tab_hbm.at[idxr_v.at[j]], buf.at[q], semg[q])

    def sct(j, p):
      pltpu.async_copy(buf.at[p], acc_sp.at[idxc_v.at[j]], sems[p], add=True)

    def wait_g(q):
      pltpu.make_async_copy(tab_hbm.at[pl.ds(0, K)], buf.at[q], semg[q]).wait()

    def wait_s(p):
      pltpu.make_async_copy(buf.at[p], acc_sp.at[pl.ds(0, K)], sems[p]).wait()

    g(0, 0)
    g(1, 1)
    wait_g(0); sct(0, 0); g(2, 2)
    wait_g(1); sct(1, 1); g(3, 3)
    wait_g(2); sct(2, 2); wait_s(0); g(4, 0)
    wait_g(3); sct(3, 3); wait_s(1); g(5, 1)

    def body(t, _):
      for p in range(4):
        j = 4 * t + p
        q = (p + 2) % 4
        wait_g(p)
        sct(j, p)
        wait_s(q)
        g(j + 2, q)
      return 0

    lax.fori_loop(1, CHQ - 1, body, 0)

    for p in range(4):
      j = 4 * (CHQ - 1) + p
      q = (p + 2) % 4
      wait_g(p)
      sct(j, p)
      wait_s(q)
      if j + 2 < CH:
        g(j + 2, q)
    wait_s(2)
    wait_s(3)
    plsc.subcore_barrier()
    pltpu.sync_copy(acc_sp.at[pl.ds(s * RP, RP)],
                    out_hbm.at[c, pl.ds(s * RP, RP)])

  return k(row2d, col2d, table, zeros)


def _tc2(x_ref, w1_ref, degp_ref, h1_ref, h1p_ref, dis_ref):
  d = degp_ref[0] + degp_ref[1]
  deg = d[:, 0:1]
  dis = jnp.where(deg > 0, lax.rsqrt(jnp.maximum(deg, 1e-12)), 0.0)
  h1 = jnp.dot(x_ref[...], w1_ref[...], preferred_element_type=jnp.float32)
  dis_ref[...] = dis
  h1_ref[...] = h1
  h1p_ref[...] = h1 * dis


def _tc3(dis_ref, aggp_ref, h1_ref, w0_ref, b1_ref, w2_ref,
         h2_ref, h2p_ref):
  dis = dis_ref[...]
  agg = aggp_ref[0] + aggp_ref[1]
  z = dis * agg + jax.nn.sigmoid(w0_ref[...]) * h1_ref[...] + b1_ref[...]
  z = jnp.maximum(z, 0.0)
  h2 = jnp.dot(z, w2_ref[...], preferred_element_type=jnp.float32)
  h2_ref[...] = h2
  h2p_ref[...] = h2 * dis


def _tc4(dis_ref, aggp_ref, h2_ref, w0_ref, b2_ref, o_ref):
  agg = aggp_ref[0] + aggp_ref[1]
  o_ref[...] = (dis_ref[...] * agg
                + jax.nn.sigmoid(w0_ref[...]) * h2_ref[...] + b2_ref[...])


def kernel(x, edge_index, w0_1, W1, b1, w0_2, W2, b2):
  ei = edge_index.astype(jnp.int32)
  row2d = ei[0].reshape(NW * CH, K)
  col2d = ei[1].reshape(NW * CH, K)

  x_pad = jnp.pad(x, ((0, NP - N), (0, 0)))
  w0_1p = jnp.pad(w0_1, (0, NP - N)).reshape(NP, 1)
  w0_2p = jnp.pad(w0_2, (0, NP - N)).reshape(NP, 1)
  W2p = jnp.pad(W2, ((0, 0), (0, F2 - C)))
  b1r = b1.reshape(1, H)
  b2r = jnp.pad(b2, (0, F2 - C)).reshape(1, F2)
  zeros48 = jnp.zeros((NP, F2), jnp.float32)
  zeros16 = jnp.zeros((NP, H), jnp.float32)

  f32 = jnp.float32
  degp = _sc_deg(col2d, zeros16)
  h1, h1p, dis = pl.pallas_call(
      _tc2, out_shape=(jax.ShapeDtypeStruct((NP, H), f32),
                       jax.ShapeDtypeStruct((NP, H), f32),
                       jax.ShapeDtypeStruct((NP, 1), f32)))(x_pad, W1, degp)
  agg1p = _sc_agg(row2d, col2d, h1p, zeros16, H)
  h2, h2p = pl.pallas_call(
      _tc3, out_shape=(jax.ShapeDtypeStruct((NP, F2), f32),
                       jax.ShapeDtypeStruct((NP, F2), f32)))(
          dis, agg1p, h1, w0_1p, b1r, W2p)
  agg2p = _sc_agg(row2d, col2d, h2p, zeros48, F2)
  out = pl.pallas_call(
      _tc4, out_shape=jax.ShapeDtypeStruct((NP, F2), f32))(
          dis, agg2p, h2, w0_2p, b2r)
  return out[:N, :C]


# speedup vs baseline: 46.0564x; 1.0638x over previous
import functools

import jax
import jax.numpy as jnp
from jax import lax
from jax.experimental import pallas as pl
from jax.experimental.pallas import tpu as pltpu
from jax.experimental.pallas import tpu_sc as plsc

N = 10000
E = 320000
D_IN = 128
H = 16
C = 40

NP = 10240
F2 = 48

NC = 2
NS = 16
NW = NC * NS
EPW = E // NW
K = 125
CH = EPW // K
CHD = (E // NS) // K
CHQ = CH // 4
RP = NP // NS
DEG_Q = 16

_mesh = plsc.VectorSubcoreMesh(core_axis_name="c", subcore_axis_name="s")
_sc_params = pltpu.CompilerParams(use_tc_tiling_on_sc=False,
                                  needs_layout_passes=False)


def _agg_pipeline(tab, idxr_v, idxc_v, buf, acc_sp, semg, sems):
  def g(j, q):
    pltpu.async_copy(tab.at[idxr_v.at[j]], buf.at[q], semg[q])

  def sct(j, p):
    pltpu.async_copy(buf.at[p], acc_sp.at[idxc_v.at[j]], sems[p], add=True)

  def wait_g(q):
    pltpu.make_async_copy(tab.at[pl.ds(0, K)], buf.at[q], semg[q]).wait()

  def wait_s(p):
    pltpu.make_async_copy(buf.at[p], acc_sp.at[pl.ds(0, K)], sems[p]).wait()

  g(0, 0)
  g(1, 1)
  wait_g(0); sct(0, 0); g(2, 2)
  wait_g(1); sct(1, 1); g(3, 3)
  wait_g(2); sct(2, 2); wait_s(0); g(4, 0)
  wait_g(3); sct(3, 3); wait_s(1); g(5, 1)

  def body(t, _):
    for p in range(4):
      j = 4 * t + p
      q = (p + 2) % 4
      wait_g(p)
      sct(j, p)
      wait_s(q)
      g(j + 2, q)
    return 0

  lax.fori_loop(1, CHQ - 1, body, 0)

  for p in range(4):
    j = 4 * (CHQ - 1) + p
    q = (p + 2) % 4
    wait_g(p)
    sct(j, p)
    wait_s(q)
    if j + 2 < CH:
      g(j + 2, q)
  wait_s(2)
  wait_s(3)


def _sc_fused_layer1(row2d, col2d, h1, zeros16):
  @functools.partial(
      pl.kernel,
      out_type=(jax.ShapeDtypeStruct((NC, NP, H), jnp.float32),
                jax.ShapeDtypeStruct((NC, NP, H), jnp.float32)),
      mesh=_mesh,
      compiler_params=_sc_params,
      scratch_types=[
          pltpu.VMEM((CHD, K), jnp.int32),
          pltpu.VMEM((CH, K), jnp.int32),
          pltpu.VMEM((CH, K), jnp.int32),
          pltpu.VMEM((4, K, H), jnp.float32),
          pltpu.VMEM((RP, H), jnp.float32),
          pltpu.VMEM((RP, H), jnp.float32),
          pltpu.VMEM((RP, H), jnp.float32),
          pltpu.VMEM((RP, H), jnp.float32),
          pltpu.VMEM((RP,), jnp.float32),
          pltpu.VMEM((RP, H), jnp.float32),
          pltpu.VMEM((K, H), jnp.float32),
          pltpu.VMEM_SHARED((NP, H), jnp.float32),
          pltpu.VMEM_SHARED((NP, H), jnp.float32),
      ] + [pltpu.SemaphoreType.DMA] * 13,
  )
  def k(row_hbm, col_hbm, h1_hbm, z_hbm, agg_out, dis_out,
        idxd, idxr_v, idxc_v, buf, degst, h1v, h1pv, disrow, disf,
        zerov, onesv, acc_sp, h1p_sp, *sems13):
    si0, si1, si2, si3, semd = sems13[0:5]
    semg = sems13[5:9]
    sems = sems13[9:13]
    c = lax.axis_index("c")
    s = lax.axis_index("s")
    w = s * NC + c

    ld0 = pltpu.async_copy(col_hbm.at[pl.ds(s * CHD, CHD)], idxd, si0)
    ld1 = pltpu.async_copy(row_hbm.at[pl.ds(w * CH, CH)], idxr_v, si1)
    ld2 = pltpu.async_copy(col_hbm.at[pl.ds(w * CH, CH)], idxc_v, si2)
    ld3 = pltpu.async_copy(h1_hbm.at[pl.ds(s * RP, RP)], h1v, si3)

    def fill_ones(i, _):
      onesv[i, :] = jnp.ones((H,), jnp.float32)
      return 0

    lax.fori_loop(0, K, fill_ones, 0)
    pltpu.sync_copy(z_hbm.at[pl.ds(s * RP, RP)], zerov)
    pltpu.sync_copy(zerov, acc_sp.at[pl.ds(s * RP, RP)])
    ld0.wait()
    plsc.subcore_barrier()

    def wait_one_deg():
      pltpu.make_async_copy(onesv, acc_sp.at[pl.ds(0, K)], semd).wait()

    def dbody(j, _):
      pltpu.async_copy(onesv, acc_sp.at[idxd.at[j]], semd, add=True)

      @pl.when(j >= DEG_Q)
      def _():
        wait_one_deg()

      return 0

    lax.fori_loop(0, CHD, dbody, 0)

    def drain(j, _):
      wait_one_deg()
      return 0

    lax.fori_loop(0, DEG_Q, drain, 0)
    plsc.subcore_barrier()

    pltpu.sync_copy(acc_sp.at[pl.ds(s * RP, RP)], degst)
    iota = lax.iota(jnp.int32, 16)
    zi = jnp.zeros((16,), jnp.int32)

    def disbody(i, _):
      v = plsc.load_gather(degst, [i * 16 + iota, zi])
      bits = lax.bitcast_convert_type(v, jnp.int32)
      y = lax.bitcast_convert_type(jnp.int32(0x5F3759DF) - (bits >> 1),
                                   jnp.float32)
      for _ in range(4):
        y = y * (1.5 - 0.5 * v * y * y)
      y = jnp.where(v > 0, y, 0.0)
      disf[pl.ds(i * 16, 16)] = y
      return 0

    lax.fori_loop(0, RP // 16, disbody, 0)

    ld3.wait()

    def scale(r, _):
      db = plsc.load_gather(disf, [jnp.full((16,), r, jnp.int32)])
      h1pv[r, :] = h1v[r, :] * db
      disrow[r, :] = db
      return 0

    lax.fori_loop(0, RP, scale, 0)
    pltpu.sync_copy(h1pv, h1p_sp.at[pl.ds(s * RP, RP)])
    pltpu.sync_copy(disrow, dis_out.at[c, pl.ds(s * RP, RP)])
    ld1.wait()
    ld2.wait()
    plsc.subcore_barrier()
    pltpu.sync_copy(zerov, acc_sp.at[pl.ds(s * RP, RP)])
    plsc.subcore_barrier()

    _agg_pipeline(h1p_sp, idxr_v, idxc_v, buf, acc_sp, semg, sems)
    plsc.subcore_barrier()
    pltpu.sync_copy(acc_sp.at[pl.ds(s * RP, RP)],
                    agg_out.at[c, pl.ds(s * RP, RP)])

  return k(row2d, col2d, h1, zeros16)


def _sc_agg(row2d, col2d, table, zeros, f):
  @functools.partial(
      pl.kernel,
      out_type=jax.ShapeDtypeStruct((NC, NP, f), jnp.float32),
      mesh=_mesh,
      compiler_params=_sc_params,
      scratch_types=[
          pltpu.VMEM((CH, K), jnp.int32),
          pltpu.VMEM((CH, K), jnp.int32),
          pltpu.VMEM((4, K, f), jnp.float32),
          pltpu.VMEM((RP, f), jnp.float32),
          pltpu.VMEM_SHARED((NP, f), jnp.float32),
      ] + [pltpu.SemaphoreType.DMA] * 8,
  )
  def k(row_hbm, col_hbm, tab_hbm, z_hbm, out_hbm,
        idxr_v, idxc_v, buf, zero_v, acc_sp, *sems8):
    semg = sems8[0:4]
    sems = sems8[4:8]
    c = lax.axis_index("c")
    s = lax.axis_index("s")
    w = s * NC + c

    ldr = pltpu.async_copy(row_hbm.at[pl.ds(w * CH, CH)], idxr_v, semg[0])
    ldc = pltpu.async_copy(col_hbm.at[pl.ds(w * CH, CH)], idxc_v, semg[1])
    pltpu.sync_copy(z_hbm.at[pl.ds(s * RP, RP)], zero_v)
    pltpu.sync_copy(zero_v, acc_sp.at[pl.ds(s * RP, RP)])
    ldr.wait()
    ldc.wait()
    plsc.subcore_barrier()

    _agg_pipeline(tab_hbm, idxr_v, idxc_v, buf, acc_sp, semg, sems)
    plsc.subcore_barrier()
    pltpu.sync_copy(acc_sp.at[pl.ds(s * RP, RP)],
                    out_hbm.at[c, pl.ds(s * RP, RP)])

  return k(row2d, col2d, table, zeros)


def _tc1(x_ref, w1_ref, h1_ref):
  h1_ref[pl.ds(0, N), :] = jnp.dot(x_ref[...], w1_ref[...],
                                   preferred_element_type=jnp.float32)
  h1_ref[pl.ds(N, NP - N), :] = jnp.zeros((NP - N, H), jnp.float32)


def _tc3(disb_ref, aggp_ref, h1_ref, w0_ref, b1_ref, w2_ref,
         h2_ref, h2p_ref):
  dis16 = disb_ref[0]
  agg = aggp_ref[0] + aggp_ref[1]
  z = dis16 * agg + jax.nn.sigmoid(w0_ref[...]) * h1_ref[...] + b1_ref[...]
  z = jnp.maximum(z, 0.0)
  h2 = jnp.dot(z, w2_ref[...], preferred_element_type=jnp.float32)
  h2_ref[...] = h2
  h2p_ref[...] = h2 * dis16[:, 0:1]


def _tc4(disb_ref, aggp_ref, h2_ref, w0_ref, b2_ref, o_ref):
  dis1 = disb_ref[0][:, 0:1]
  agg = aggp_ref[0] + aggp_ref[1]
  out = dis1 * agg + jax.nn.sigmoid(w0_ref[...]) * h2_ref[...] + b2_ref[...]
  o_ref[...] = out[:N, :C]


def kernel(x, edge_index, w0_1, W1, b1, w0_2, W2, b2):
  ei = edge_index.astype(jnp.int32)
  row2d = ei[0].reshape(NW * CH, K)
  col2d = ei[1].reshape(NW * CH, K)

  w0_1p = jnp.pad(w0_1, (0, NP - N)).reshape(NP, 1)
  w0_2p = jnp.pad(w0_2, (0, NP - N)).reshape(NP, 1)
  W2p = jnp.pad(W2, ((0, 0), (0, F2 - C)))
  b1r = b1.reshape(1, H)
  b2r = jnp.pad(b2, (0, F2 - C)).reshape(1, F2)
  zeros48 = jnp.zeros((NP, F2), jnp.float32)
  zeros16 = jnp.zeros((NP, H), jnp.float32)

  f32 = jnp.float32
  h1 = pl.pallas_call(
      _tc1, out_shape=jax.ShapeDtypeStruct((NP, H), f32))(x, W1)
  agg1p, disb = _sc_fused_layer1(row2d, col2d, h1, zeros16)
  h2, h2p = pl.pallas_call(
      _tc3, out_shape=(jax.ShapeDtypeStruct((NP, F2), f32),
                       jax.ShapeDtypeStruct((NP, F2), f32)))(
          disb, agg1p, h1, w0_1p, b1r, W2p)
  agg2p = _sc_agg(row2d, col2d, h2p, zeros48, F2)
  out = pl.pallas_call(
      _tc4, out_shape=jax.ShapeDtypeStruct((N, C), f32))(
          disb, agg2p, h2, w0_2p, b2r)
  return out
